# Initial kernel scaffold; baseline (speedup 1.0000x reference)
import functools

import jax
import jax.numpy as jnp
from jax import lax
from jax.experimental import pallas as pl
from jax.experimental.pallas import tpu as pltpu
from jax.experimental.pallas import tpu_sc as plsc

_VOCAB = 100000
_D = 128
_H = 512
_B = 16384

_info = plsc.get_sparse_core_info()
_NC, _NS = _info.num_cores, _info.num_subcores
_NW = _NC * _NS
_BPW = _B // _NW


def _sc_gather(qtab_hbm, etab_hbm, qids_hbm, eids_hbm, q_out, e_out,
               idx_v, rows_v, sem):
    wid = lax.axis_index("s") * _NC + lax.axis_index("c")
    base = wid * _BPW
    pltpu.sync_copy(qids_hbm.at[pl.ds(base, _BPW)], idx_v)
    pltpu.async_copy(qtab_hbm.at[idx_v], rows_v, sem).wait()
    pltpu.sync_copy(rows_v, q_out.at[pl.ds(base, _BPW)])
    pltpu.sync_copy(eids_hbm.at[pl.ds(base, _BPW)], idx_v)
    pltpu.async_copy(etab_hbm.at[idx_v], rows_v, sem).wait()
    pltpu.sync_copy(rows_v, e_out.at[pl.ds(base, _BPW)])


_gather_call = functools.partial(
    pl.kernel,
    mesh=plsc.VectorSubcoreMesh(core_axis_name="c", subcore_axis_name="s"),
    out_type=[
        jax.ShapeDtypeStruct((_B, _D), jnp.float32),
        jax.ShapeDtypeStruct((_B, _D), jnp.float32),
    ],
    scratch_types=[
        pltpu.VMEM((_BPW,), jnp.int32),
        pltpu.VMEM((_BPW, _D), jnp.float32),
        pltpu.SemaphoreType.DMA,
    ],
)(_sc_gather)


_BLK = 1024


def _mlp_body(x_ref, w1_ref, b1_ref, w2_ref, b2_ref, o_ref):
    x = x_ref[...]
    h = jnp.dot(x, w1_ref[...], preferred_element_type=jnp.float32)
    h = h + b1_ref[...]
    h = jax.nn.gelu(h)
    o = jnp.dot(h, w2_ref[...], preferred_element_type=jnp.float32)
    o_ref[...] = o + b2_ref[...]


def _mlp(x, W1, b1, W2, b2):
    n = x.shape[0]
    grid = n // _BLK
    return pl.pallas_call(
        _mlp_body,
        grid=(grid,),
        in_specs=[
            pl.BlockSpec((_BLK, _D), lambda i: (i, 0)),
            pl.BlockSpec((_D, _H), lambda i: (0, 0)),
            pl.BlockSpec((1, _H), lambda i: (0, 0)),
            pl.BlockSpec((_H, _D), lambda i: (0, 0)),
            pl.BlockSpec((1, _D), lambda i: (0, 0)),
        ],
        out_specs=pl.BlockSpec((_BLK, _D), lambda i: (i, 0)),
        out_shape=jax.ShapeDtypeStruct((n, _D), jnp.float32),
    )(x, W1, b1.reshape(1, _H), W2, b2.reshape(1, _D))


@jax.jit
def kernel(query_ids, entity_ids, query_emb, ent_emb, W1, b1, W2, b2):
    q_rows, e_rows = _gather_call(
        query_emb, ent_emb,
        query_ids.astype(jnp.int32), entity_ids.astype(jnp.int32))
    x = jnp.concatenate([q_rows, e_rows], axis=0)
    y = _mlp(x, W1, b1, W2, b2)
    return y[:_B], y[_B:]


# speedup vs baseline: 1.7347x; 1.7347x over previous
import functools

import jax
import jax.numpy as jnp
from jax import lax
from jax.experimental import pallas as pl
from jax.experimental.pallas import tpu as pltpu
from jax.experimental.pallas import tpu_sc as plsc

_VOCAB = 100000
_D = 128
_H = 512
_B = 16384

_NC, _NS = 2, 16
_NW = _NC * _NS
_BPW = _B // _NW


def _sc_gather(qtab_hbm, etab_hbm, qids_hbm, eids_hbm, q_out, e_out,
               idx_v, rows_v, sem):
    wid = lax.axis_index("s") * _NC + lax.axis_index("c")
    base = wid * _BPW
    pltpu.sync_copy(qids_hbm.at[pl.ds(base, _BPW)], idx_v)
    pltpu.async_copy(qtab_hbm.at[idx_v], rows_v, sem).wait()
    pltpu.sync_copy(rows_v, q_out.at[pl.ds(base, _BPW)])
    pltpu.sync_copy(eids_hbm.at[pl.ds(base, _BPW)], idx_v)
    pltpu.async_copy(etab_hbm.at[idx_v], rows_v, sem).wait()
    pltpu.sync_copy(rows_v, e_out.at[pl.ds(base, _BPW)])


@functools.lru_cache(maxsize=None)
def _gather_call():
    return pl.kernel(
        _sc_gather,
        mesh=plsc.VectorSubcoreMesh(core_axis_name="c", subcore_axis_name="s"),
        out_type=[
            jax.ShapeDtypeStruct((_B, _D), jnp.float32),
            jax.ShapeDtypeStruct((_B, _D), jnp.float32),
        ],
        scratch_types=[
            pltpu.VMEM((_BPW,), jnp.int32),
            pltpu.VMEM((_BPW, _D), jnp.float32),
            pltpu.SemaphoreType.DMA,
        ],
    )


_BLK = 1024


def _mlp_body(x_ref, w1_ref, b1_ref, w2_ref, b2_ref, o_ref):
    x = x_ref[...]
    h = jnp.dot(x, w1_ref[...], preferred_element_type=jnp.float32)
    h = h + b1_ref[...]
    h = jax.nn.gelu(h)
    o = jnp.dot(h, w2_ref[...], preferred_element_type=jnp.float32)
    o_ref[...] = o + b2_ref[...]


def _mlp(x, W1, b1, W2, b2):
    n = x.shape[0]
    grid = n // _BLK
    return pl.pallas_call(
        _mlp_body,
        grid=(grid,),
        in_specs=[
            pl.BlockSpec((_BLK, _D), lambda i: (i, 0)),
            pl.BlockSpec((_D, _H), lambda i: (0, 0)),
            pl.BlockSpec((1, _H), lambda i: (0, 0)),
            pl.BlockSpec((_H, _D), lambda i: (0, 0)),
            pl.BlockSpec((1, _D), lambda i: (0, 0)),
        ],
        out_specs=pl.BlockSpec((_BLK, _D), lambda i: (i, 0)),
        out_shape=jax.ShapeDtypeStruct((n, _D), jnp.float32),
    )(x, W1, b1.reshape(1, _H), W2, b2.reshape(1, _D))


@jax.jit
def kernel(query_ids, entity_ids, query_emb, ent_emb, W1, b1, W2, b2):
    q_rows, e_rows = _gather_call()(
        query_emb, ent_emb,
        query_ids.astype(jnp.int32), entity_ids.astype(jnp.int32))
    x = jnp.concatenate([q_rows, e_rows], axis=0)
    y = _mlp(x, W1, b1, W2, b2)
    return y[:_B], y[_B:]


# speedup vs baseline: 2.5634x; 1.4777x over previous
import functools

import jax
import jax.numpy as jnp
from jax import lax
from jax.experimental import pallas as pl
from jax.experimental.pallas import tpu as pltpu
from jax.experimental.pallas import tpu_sc as plsc

_VOCAB = 100000
_D = 128
_H = 512
_B = 16384

_NC, _NS = 2, 16
_NW = _NC * _NS
_BPW = _B // _NW


def _sc_gather(qtab_hbm, etab_hbm, qids_hbm, eids_hbm, q_out, e_out,
               idx_v, rows_v, sem):
    wid = lax.axis_index("s") * _NC + lax.axis_index("c")
    base = wid * _BPW
    pltpu.sync_copy(qids_hbm.at[pl.ds(base, _BPW)], idx_v)
    pltpu.async_copy(qtab_hbm.at[idx_v], rows_v, sem).wait()
    pltpu.sync_copy(rows_v, q_out.at[pl.ds(base, _BPW)])
    pltpu.sync_copy(eids_hbm.at[pl.ds(base, _BPW)], idx_v)
    pltpu.async_copy(etab_hbm.at[idx_v], rows_v, sem).wait()
    pltpu.sync_copy(rows_v, e_out.at[pl.ds(base, _BPW)])


@functools.lru_cache(maxsize=None)
def _gather_call():
    return pl.kernel(
        _sc_gather,
        mesh=plsc.VectorSubcoreMesh(core_axis_name="c", subcore_axis_name="s"),
        out_type=[
            jax.ShapeDtypeStruct((_B, _D), jnp.float32),
            jax.ShapeDtypeStruct((_B, _D), jnp.float32),
        ],
        scratch_types=[
            pltpu.VMEM((_BPW,), jnp.int32),
            pltpu.VMEM((_BPW, _D), jnp.float32),
            pltpu.SemaphoreType.DMA,
        ],
    )


_BLK = 1024


def _mlp_body(xq_ref, xe_ref, w1_ref, b1_ref, w2_ref, b2_ref,
              oq_ref, oe_ref):
    w1 = w1_ref[...]
    w2 = w2_ref[...]
    b1 = b1_ref[...]
    b2 = b2_ref[...]
    hq = jax.nn.gelu(
        jnp.dot(xq_ref[...], w1, preferred_element_type=jnp.float32) + b1)
    oq_ref[...] = jnp.dot(hq, w2, preferred_element_type=jnp.float32) + b2
    he = jax.nn.gelu(
        jnp.dot(xe_ref[...], w1, preferred_element_type=jnp.float32) + b1)
    oe_ref[...] = jnp.dot(he, w2, preferred_element_type=jnp.float32) + b2


def _mlp2(xq, xe, W1, b1, W2, b2):
    grid = _B // _BLK
    row_spec = pl.BlockSpec((_BLK, _D), lambda i: (i, 0))
    return pl.pallas_call(
        _mlp_body,
        grid=(grid,),
        in_specs=[
            row_spec,
            row_spec,
            pl.BlockSpec((_D, _H), lambda i: (0, 0)),
            pl.BlockSpec((1, _H), lambda i: (0, 0)),
            pl.BlockSpec((_H, _D), lambda i: (0, 0)),
            pl.BlockSpec((1, _D), lambda i: (0, 0)),
        ],
        out_specs=[row_spec, row_spec],
        out_shape=[
            jax.ShapeDtypeStruct((_B, _D), jnp.float32),
            jax.ShapeDtypeStruct((_B, _D), jnp.float32),
        ],
    )(xq, xe, W1, b1.reshape(1, _H), W2, b2.reshape(1, _D))


@jax.jit
def kernel(query_ids, entity_ids, query_emb, ent_emb, W1, b1, W2, b2):
    q_rows, e_rows = _gather_call()(
        query_emb, ent_emb,
        query_ids.astype(jnp.int32), entity_ids.astype(jnp.int32))
    return _mlp2(q_rows, e_rows, W1, b1, W2, b2)
